# baseline (speedup 1.0000x reference)
import functools

import jax
import jax.numpy as jnp
from jax import lax
from jax.experimental import pallas as pl
from jax.experimental.pallas import tpu as pltpu
from jax.experimental.pallas import tpu_sc as plsc

R = 16384
C = 1000
K = 200
NC, NS, L = 2, 16, 16
NW = NC * NS
RPW = R // NW
B = 32
NB = RPW // B
PAIRS = B // 2
CHUNKS = (2 * K) // L


@functools.partial(
    pl.kernel,
    mesh=plsc.VectorSubcoreMesh(core_axis_name="c", subcore_axis_name="s"),
    compiler_params=pltpu.CompilerParams(needs_layout_passes=False),
    out_type=jax.ShapeDtypeStruct((R * K,), jnp.float32),
    scratch_types=[
        pltpu.VMEM((B * C,), jnp.float32),
        pltpu.VMEM((B * K,), jnp.int32),
        pltpu.VMEM((B * K,), jnp.float32),
        pltpu.VMEM((2 * K,), jnp.int32),
    ],
)
def _gather_kernel(x_hbm, ind_hbm, roff_hbm, out_hbm, x_v, ind_v, out_v, roff_v):
    wid = lax.axis_index("s") * NC + lax.axis_index("c")
    base_row = wid * RPW

    pltpu.sync_copy(roff_hbm, roff_v)

    def do_block(b, _):
        row0 = base_row + b * B
        pltpu.sync_copy(x_hbm.at[pl.ds(row0 * C, B * C)], x_v)
        pltpu.sync_copy(ind_hbm.at[pl.ds(row0 * K, B * K)], ind_v)

        def do_pair(g, _):
            pair_base = jnp.full((L,), g * (2 * C), jnp.int32)
            out0 = g * (2 * K)
            for c in range(CHUNKS):
                col = ind_v[pl.ds(out0 + c * L, L)]
                fidx = col + (pair_base + roff_v[pl.ds(c * L, L)])
                out_v[pl.ds(out0 + c * L, L)] = plsc.load_gather(x_v, [fidx])
            return 0

        lax.fori_loop(0, PAIRS, do_pair, 0, unroll=False)
        pltpu.sync_copy(out_v, out_hbm.at[pl.ds(row0 * K, B * K)])
        return 0

    lax.fori_loop(0, NB, do_block, 0, unroll=False)


def kernel(x, ind1):
    roff = jnp.where(jnp.arange(2 * K, dtype=jnp.int32) >= K, C, 0).astype(jnp.int32)
    out = _gather_kernel(x.reshape(-1), ind1.reshape(-1), roff)
    return (out.reshape(R, K),)


# speedup vs baseline: 1.7879x; 1.7879x over previous
import functools

import jax
import jax.numpy as jnp
from jax import lax
from jax.experimental import pallas as pl
from jax.experimental.pallas import tpu as pltpu
from jax.experimental.pallas import tpu_sc as plsc

R = 16384
C = 1000
K = 200
NC, NS, L = 2, 16, 16
NW = NC * NS
RPW = R // NW
B = 32
NB = RPW // B
FULL = K // L
LAST = K - L


@functools.partial(
    pl.kernel,
    mesh=plsc.VectorSubcoreMesh(core_axis_name="c", subcore_axis_name="s"),
    compiler_params=pltpu.CompilerParams(needs_layout_passes=False),
    out_type=jax.ShapeDtypeStruct((R, K), jnp.float32),
    scratch_types=[
        pltpu.VMEM((B, C), jnp.float32),
        pltpu.VMEM((B, K), jnp.int32),
        pltpu.VMEM((B, K), jnp.float32),
    ],
)
def _gather_kernel(x_hbm, ind_hbm, out_hbm, x_v, ind_v, out_v):
    wid = lax.axis_index("s") * NC + lax.axis_index("c")
    base_row = wid * RPW

    def do_block(b, _):
        row0 = base_row + b * B
        pltpu.sync_copy(x_hbm.at[pl.ds(row0, B)], x_v)
        pltpu.sync_copy(ind_hbm.at[pl.ds(row0, B)], ind_v)

        def do_row(r, _):
            row = jnp.full((L,), r, jnp.int32)
            for c in range(FULL):
                col = ind_v[r, pl.ds(c * L, L)]
                out_v[r, pl.ds(c * L, L)] = plsc.load_gather(x_v, [row, col])
            col = ind_v[r, pl.ds(LAST, L)]
            out_v[r, pl.ds(LAST, L)] = plsc.load_gather(x_v, [row, col])
            return 0

        lax.fori_loop(0, B, do_row, 0, unroll=False)
        pltpu.sync_copy(out_v, out_hbm.at[pl.ds(row0, B)])
        return 0

    lax.fori_loop(0, NB, do_block, 0, unroll=False)


def kernel(x, ind1):
    return (_gather_kernel(x, ind1),)
